# baseline (speedup 1.0000x reference)
import jax
import jax.numpy as jnp
from jax import lax
from jax.experimental import pallas as pl
from jax.experimental.pallas import tpu as pltpu
from jax.experimental.pallas import tpu_sc as plsc

W = 400
H = 400
D = 12
T = 5
HW = H * W
DHW = D * HW
NPTS = T * 100000
GRID = T * DHW
HALF = GRID // 2
PADW = 512
RSZ = HALF + PADW
PGRID = 2 * RSZ

NTILES = 16
TILE_PTS = 31256
STAGE_PTS = 2048
NSTAGES = 16
GROUPS = STAGE_PTS // 16
ROW = 128
MAXROWS = NSTAGES * STAGE_PTS // ROW + 2
ZCH = 9376
ZN = RSZ // NTILES // ZCH


def _body(xs_h, ys_h, zs_h, out, pbx0, pbx1, pby0, pby1, pbz0, pbz1,
          ibuf, sbuf, zbuf, ones, zsem, psem, ssem):
    pbx = (pbx0, pbx1)
    pby = (pby0, pby1)
    pbz = (pbz0, pbz1)
    cid = lax.axis_index("c")
    wid = lax.axis_index("s")
    chigh = cid == 1
    iota = lax.iota(jnp.int32, 16)
    zvec = jnp.zeros((16,), jnp.float32)
    onev = jnp.full((16,), 1.0, jnp.float32)
    trashv = cid * RSZ + HALF + wid * 32 + iota

    def _zb(i, c):
        zbuf[pl.ds(i * 16, 16)] = zvec
        return c

    with jax.named_scope("ph0_init"):
        lax.fori_loop(0, ZCH // 16, _zb, 0)
        for i in range(ROW // 16):
            ones[pl.ds(i * 16, 16)] = onev

    zbase = cid * RSZ + wid * (RSZ // NTILES)

    def _zfire(k, c):
        pltpu.async_copy(zbuf, out.at[pl.ds(zbase + k * ZCH, ZCH)], zsem)
        return c

    with jax.named_scope("ph1_zfire"):
        lax.fori_loop(0, ZN, _zfire, 0)

    base = wid * TILE_PTS

    def _sstart(s):
        return jnp.minimum(base + s * STAGE_PTS, NPTS - STAGE_PTS)

    def _pt_copies(s):
        sl = pl.ds(_sstart(s), STAGE_PTS)
        b = s % 2
        return (pltpu.make_async_copy(xs_h.at[sl], pbx[b], psem),
                pltpu.make_async_copy(ys_h.at[sl], pby[b], psem),
                pltpu.make_async_copy(zs_h.at[sl], pbz[b], psem))

    def _flush(args):
        pos, row = args
        for k in range(ROW // 16):
            ibuf[row, pl.ds(k * 16, 16)] = sbuf[pl.ds(k * 16, 16)]
        sbuf[pl.ds(0, 16)] = sbuf[pl.ds(ROW, 16)]
        return pos - ROW, row + 1

    for cp in _pt_copies(0):
        cp.start()
    pos = jnp.int32(0)
    row = jnp.int32(0)
    for s in range(NSTAGES):
      with jax.named_scope("ph2_stage"):
        if s + 1 < NSTAGES:
            for cp in _pt_copies(s + 1):
                cp.start()
        with jax.named_scope("ph2_wait"):
            for cp in _pt_copies(s):
                cp.wait()
        bx, by, bz = pbx[s % 2], pby[s % 2], pbz[s % 2]
        sp = _sstart(s)

        def _grp(g, carry, s=s, bx=bx, by=by, bz=bz, sp=sp):
            pos, row = carry
            o = pl.ds(g * 16, 16)
            x = bx[o]
            y = by[o]
            z = bz[o]
            tw = (x + 50.0) * 4.0
            th = (y + 50.0) * 4.0
            td = (z + 3.0) * 2.0
            iw = jnp.minimum(jnp.maximum(tw, -1.0), 512.0).astype(jnp.int32)
            ih = jnp.minimum(jnp.maximum(th, -1.0), 512.0).astype(jnp.int32)
            idd = jnp.minimum(jnp.maximum(td, -1.0), 64.0).astype(jnp.int32)
            valid = ((tw >= 0.0) & (th >= 0.0) & (td >= 0.0)
                     & (iw < W) & (ih < H) & (idd < D))
            pid = sp + g * 16 + iota
            tpl = (jnp.where(pid >= 100000, DHW, 0)
                   + jnp.where(pid >= 200000, DHW, 0)
                   + jnp.where(pid >= 300000, DHW, 0)
                   + jnp.where(pid >= 400000, DHW, 0))
            flat = tpl + idd * HW + ih * W + iw
            mhigh = flat >= HALF
            off = flat + jnp.where(mhigh, PADW, 0)
            keep = valid & (mhigh == chigh)
            plsc.store_compressed(sbuf.at[pl.ds(pos, 16)], off, mask=keep)
            pos = pos + jnp.sum(keep.astype(jnp.int32))
            return lax.cond(pos >= ROW, _flush, lambda a: a, (pos, row))

        with jax.named_scope("ph2_compute"):
            pos, row = lax.fori_loop(0, GROUPS, _grp, (pos, row))

    with jax.named_scope("ph2_tail"):
        for k in range(ROW // 16):
            sbuf[pl.ds(pos + k * 16, 16)] = trashv
        _, row = lax.cond(pos > 0,
                          lambda a: _flush(a),
                          lambda a: a, (pos, row))

    def _zdrain(k, c):
        pltpu.make_async_copy(zbuf, out.at[pl.ds(zbase + k * ZCH, ZCH)],
                              zsem).wait()
        return c

    with jax.named_scope("ph3_zdrain"):
        lax.fori_loop(0, ZN, _zdrain, 0)
    with jax.named_scope("ph4_barrier"):
        plsc.subcore_barrier()

    def _sfire(r, c):
        pltpu.async_copy(ones, out.at[ibuf.at[r]], ssem)
        return c

    with jax.named_scope("ph5_sfire"):
        lax.fori_loop(0, row, _sfire, 0)

    def _sdrain(r, c):
        pltpu.make_async_copy(ones, out.at[ibuf.at[0]], ssem).wait()
        return c

    with jax.named_scope("ph6_sdrain"):
        lax.fori_loop(0, row, _sdrain, 0)


@jax.jit
def _voxelize(xs, ys, zs):
    mesh = plsc.VectorSubcoreMesh(core_axis_name="c", subcore_axis_name="s")
    grid = pl.kernel(
        _body,
        out_type=jax.ShapeDtypeStruct((PGRID,), jnp.float32),
        mesh=mesh,
        compiler_params=pltpu.CompilerParams(needs_layout_passes=False),
        scratch_types=(
            [pltpu.VMEM((STAGE_PTS,), jnp.float32) for _ in range(6)]
            + [
                pltpu.VMEM((MAXROWS, ROW), jnp.int32),
                pltpu.VMEM((2 * ROW + 32,), jnp.int32),
                pltpu.VMEM((ZCH,), jnp.float32),
                pltpu.VMEM((ROW,), jnp.float32),
                pltpu.SemaphoreType.DMA,
                pltpu.SemaphoreType.DMA,
                pltpu.SemaphoreType.DMA,
            ]
        ),
    )(xs, ys, zs)
    lo = grid[:HALF]
    hi = grid[RSZ:RSZ + HALF]
    return jnp.maximum(
        jnp.concatenate([lo, hi]).reshape(T * D, H, W), 0.0)


def kernel(lidars):
    pts = lidars[0]
    xs = pts[:, :, 0].reshape(-1)
    ys = pts[:, :, 1].reshape(-1)
    zs = pts[:, :, 2].reshape(-1)
    return _voxelize(xs, ys, zs)


# speedup vs baseline: 2.8689x; 2.8689x over previous
import jax
import jax.numpy as jnp
from jax import lax
from jax.experimental import pallas as pl
from jax.experimental.pallas import tpu as pltpu
from jax.experimental.pallas import tpu_sc as plsc

W = 400
H = 400
D = 12
T = 5
HW = H * W
DHW = D * HW
NPTS = T * 100000
GRID = T * DHW
HALF = GRID // 2
PADW = 512
RSZ = HALF + PADW
PGRID = 2 * RSZ

NTILES = 16
TILE_PTS = 31256
STAGE_PTS = 2048
NSTAGES = 16
GROUPS = STAGE_PTS // 16
ROW = 128
MAXROWS = NSTAGES * STAGE_PTS // ROW + 2
ZCH = 9376
ZN = RSZ // NTILES // ZCH
HTSZ = 32768


def _body(xs_h, ys_h, zs_h, out, pbx0, pbx1, pby0, pby1, pbz0, pbz1,
          ibuf, sbuf, zbuf, ones, ht, zsem, psem, ssem):
    pbx = (pbx0, pbx1)
    pby = (pby0, pby1)
    pbz = (pbz0, pbz1)
    cid = lax.axis_index("c")
    wid = lax.axis_index("s")
    chigh = cid == 1
    iota = lax.iota(jnp.int32, 16)
    zvec = jnp.zeros((16,), jnp.float32)
    onev = jnp.full((16,), 1.0, jnp.float32)
    trashv = cid * RSZ + HALF + wid * 32 + iota

    def _zb(i, c):
        zbuf[pl.ds(i * 16, 16)] = zvec
        return c

    sent = jnp.full((16,), -1, jnp.int32)

    def _hti(i, c):
        ht[pl.ds(i * 16, 16)] = sent
        return c

    with jax.named_scope("ph0_init"):
        lax.fori_loop(0, ZCH // 16, _zb, 0)
        for i in range(ROW // 16):
            ones[pl.ds(i * 16, 16)] = onev
        lax.fori_loop(0, HTSZ // 16, _hti, 0)

    zbase = cid * RSZ + wid * (RSZ // NTILES)

    def _zfire(k, c):
        pltpu.async_copy(zbuf, out.at[pl.ds(zbase + k * ZCH, ZCH)], zsem)
        return c

    with jax.named_scope("ph1_zfire"):
        lax.fori_loop(0, ZN, _zfire, 0)

    base = wid * TILE_PTS

    def _sstart(s):
        return jnp.minimum(base + s * STAGE_PTS, NPTS - STAGE_PTS)

    def _pt_copies(s):
        sl = pl.ds(_sstart(s), STAGE_PTS)
        b = s % 2
        return (pltpu.make_async_copy(xs_h.at[sl], pbx[b], psem),
                pltpu.make_async_copy(ys_h.at[sl], pby[b], psem),
                pltpu.make_async_copy(zs_h.at[sl], pbz[b], psem))

    def _flush(args):
        pos, row = args
        for k in range(ROW // 16):
            ibuf[row, pl.ds(k * 16, 16)] = sbuf[pl.ds(k * 16, 16)]
        sbuf[pl.ds(0, 16)] = sbuf[pl.ds(ROW, 16)]
        return pos - ROW, row + 1

    for cp in _pt_copies(0):
        cp.start()
    pos = jnp.int32(0)
    row = jnp.int32(0)
    for s in range(NSTAGES):
      with jax.named_scope("ph2_stage"):
        if s + 1 < NSTAGES:
            for cp in _pt_copies(s + 1):
                cp.start()
        with jax.named_scope("ph2_wait"):
            for cp in _pt_copies(s):
                cp.wait()
        bx, by, bz = pbx[s % 2], pby[s % 2], pbz[s % 2]
        sp = _sstart(s)

        def _grp(g, carry, s=s, bx=bx, by=by, bz=bz, sp=sp):
            pos, row = carry
            o = pl.ds(g * 16, 16)
            x = bx[o]
            y = by[o]
            z = bz[o]
            tw = (x + 50.0) * 4.0
            th = (y + 50.0) * 4.0
            td = (z + 3.0) * 2.0
            iw = jnp.minimum(jnp.maximum(tw, -1.0), 512.0).astype(jnp.int32)
            ih = jnp.minimum(jnp.maximum(th, -1.0), 512.0).astype(jnp.int32)
            idd = jnp.minimum(jnp.maximum(td, -1.0), 64.0).astype(jnp.int32)
            valid = ((tw >= 0.0) & (th >= 0.0) & (td >= 0.0)
                     & (iw < W) & (ih < H) & (idd < D))
            pid = sp + g * 16 + iota
            tpl = (jnp.where(pid >= 100000, DHW, 0)
                   + jnp.where(pid >= 200000, DHW, 0)
                   + jnp.where(pid >= 300000, DHW, 0)
                   + jnp.where(pid >= 400000, DHW, 0))
            flat = tpl + idd * HW + ih * W + iw
            mhigh = flat >= HALF
            off = flat + jnp.where(mhigh, PADW, 0)
            keep = valid & (mhigh == chigh)
            h = (off ^ lax.shift_right_logical(off, 13)) & (HTSZ - 1)
            got = plsc.load_gather(ht, [h], mask=keep)
            plsc.store_scatter(ht, [h], off, mask=keep)
            keep = keep & (got != off)
            plsc.store_compressed(sbuf.at[pl.ds(pos, 16)], off, mask=keep)
            pos = pos + jnp.sum(keep.astype(jnp.int32))
            return lax.cond(pos >= ROW, _flush, lambda a: a, (pos, row))

        with jax.named_scope("ph2_compute"):
            pos, row = lax.fori_loop(0, GROUPS, _grp, (pos, row))

    with jax.named_scope("ph2_tail"):
        for k in range(ROW // 16):
            sbuf[pl.ds(pos + k * 16, 16)] = trashv
        _, row = lax.cond(pos > 0,
                          lambda a: _flush(a),
                          lambda a: a, (pos, row))

    def _zdrain(k, c):
        pltpu.make_async_copy(zbuf, out.at[pl.ds(zbase + k * ZCH, ZCH)],
                              zsem).wait()
        return c

    with jax.named_scope("ph3_zdrain"):
        lax.fori_loop(0, ZN, _zdrain, 0)
    with jax.named_scope("ph4_barrier"):
        plsc.subcore_barrier()

    def _sfire(r, c):
        pltpu.async_copy(ones, out.at[ibuf.at[r]], ssem)
        return c

    with jax.named_scope("ph5_sfire"):
        lax.fori_loop(0, row, _sfire, 0)

    def _sdrain(r, c):
        pltpu.make_async_copy(ones, out.at[ibuf.at[0]], ssem).wait()
        return c

    with jax.named_scope("ph6_sdrain"):
        lax.fori_loop(0, row, _sdrain, 0)


@jax.jit
def _voxelize(xs, ys, zs):
    mesh = plsc.VectorSubcoreMesh(core_axis_name="c", subcore_axis_name="s")
    grid = pl.kernel(
        _body,
        out_type=jax.ShapeDtypeStruct((PGRID,), jnp.float32),
        mesh=mesh,
        compiler_params=pltpu.CompilerParams(needs_layout_passes=False),
        scratch_types=(
            [pltpu.VMEM((STAGE_PTS,), jnp.float32) for _ in range(6)]
            + [
                pltpu.VMEM((MAXROWS, ROW), jnp.int32),
                pltpu.VMEM((2 * ROW + 32,), jnp.int32),
                pltpu.VMEM((ZCH,), jnp.float32),
                pltpu.VMEM((ROW,), jnp.float32),
                pltpu.VMEM((HTSZ,), jnp.int32),
                pltpu.SemaphoreType.DMA,
                pltpu.SemaphoreType.DMA,
                pltpu.SemaphoreType.DMA,
            ]
        ),
    )(xs, ys, zs)
    lo = grid[:HALF]
    hi = grid[RSZ:RSZ + HALF]
    return jnp.maximum(
        jnp.concatenate([lo, hi]).reshape(T * D, H, W), 0.0)


def kernel(lidars):
    pts = lidars[0]
    xs = pts[:, :, 0].reshape(-1)
    ys = pts[:, :, 1].reshape(-1)
    zs = pts[:, :, 2].reshape(-1)
    return _voxelize(xs, ys, zs)


# speedup vs baseline: 2.9388x; 1.0244x over previous
import jax
import jax.numpy as jnp
from jax import lax
from jax.experimental import pallas as pl
from jax.experimental.pallas import tpu as pltpu
from jax.experimental.pallas import tpu_sc as plsc

W = 400
H = 400
D = 12
T = 5
HW = H * W
DHW = D * HW
NPTS = T * 100000
GRID = T * DHW
HALF = GRID // 2
PADW = 512
RSZ = HALF + PADW
PGRID = 2 * RSZ

NTILES = 16
TILE_PTS = 31256
STAGE_PTS = 2048
NSTAGES = 16
GROUPS = STAGE_PTS // 16
ROW = 128
MAXROWS = NSTAGES * STAGE_PTS // ROW + 2
ZCH = 18752
ZN = RSZ // NTILES // ZCH
HTSZ = 32768


def _body(xs_h, ys_h, zs_h, out, pbx0, pbx1, pby0, pby1, pbz0, pbz1,
          ibuf, sbuf, zbuf, ones, ht, zsem, psem, ssem):
    pbx = (pbx0, pbx1)
    pby = (pby0, pby1)
    pbz = (pbz0, pbz1)
    cid = lax.axis_index("c")
    wid = lax.axis_index("s")
    chigh = cid == 1
    iota = lax.iota(jnp.int32, 16)
    zvec = jnp.zeros((16,), jnp.float32)
    onev = jnp.full((16,), 1.0, jnp.float32)
    trashv = cid * RSZ + HALF + wid * 32 + iota

    def _zb(i, c):
        zbuf[pl.ds(i * 16, 16)] = zvec
        return c

    sent = jnp.full((16,), -1, jnp.int32)

    def _hti(i, c):
        for k in range(4):
            ht[pl.ds(i * 64 + k * 16, 16)] = sent
        return c

    with jax.named_scope("ph0_init"):
        lax.fori_loop(0, ZCH // 16, _zb, 0)
        for i in range(ROW // 16):
            ones[pl.ds(i * 16, 16)] = onev
        lax.fori_loop(0, HTSZ // 64, _hti, 0)

    zbase = cid * RSZ + wid * (RSZ // NTILES)

    def _zfire(k, c):
        pltpu.async_copy(zbuf, out.at[pl.ds(zbase + k * ZCH, ZCH)], zsem)
        return c

    with jax.named_scope("ph1_zfire"):
        lax.fori_loop(0, ZN, _zfire, 0)

    base = wid * TILE_PTS

    def _sstart(s):
        return jnp.minimum(base + s * STAGE_PTS, NPTS - STAGE_PTS)

    def _pt_copies(s):
        sl = pl.ds(_sstart(s), STAGE_PTS)
        b = s % 2
        return (pltpu.make_async_copy(xs_h.at[sl], pbx[b], psem),
                pltpu.make_async_copy(ys_h.at[sl], pby[b], psem),
                pltpu.make_async_copy(zs_h.at[sl], pbz[b], psem))

    def _flush(args):
        pos, row = args
        for k in range(ROW // 16):
            ibuf[row, pl.ds(k * 16, 16)] = sbuf[pl.ds(k * 16, 16)]
        sbuf[pl.ds(0, 16)] = sbuf[pl.ds(ROW, 16)]
        return pos - ROW, row + 1

    for cp in _pt_copies(0):
        cp.start()
    pos = jnp.int32(0)
    row = jnp.int32(0)
    for s in range(NSTAGES):
      with jax.named_scope("ph2_stage"):
        if s + 1 < NSTAGES:
            for cp in _pt_copies(s + 1):
                cp.start()
        with jax.named_scope("ph2_wait"):
            for cp in _pt_copies(s):
                cp.wait()
        bx, by, bz = pbx[s % 2], pby[s % 2], pbz[s % 2]
        sp = _sstart(s)
        t0 = ((sp >= 100000).astype(jnp.int32)
              + (sp >= 200000).astype(jnp.int32)
              + (sp >= 300000).astype(jnp.int32)
              + (sp >= 400000).astype(jnp.int32))
        tplb = t0 * DHW
        thr = t0 * 100000 + 100000

        def _grp(g, carry, s=s, bx=bx, by=by, bz=bz, sp=sp,
                 tplb=tplb, thr=thr):
            pos, row = carry
            o = pl.ds(g * 16, 16)
            x = bx[o]
            y = by[o]
            z = bz[o]
            tw = (x + 50.0) * 4.0
            th = (y + 50.0) * 4.0
            td = (z + 3.0) * 2.0
            iw = tw.astype(jnp.int32)
            ih = th.astype(jnp.int32)
            idd = td.astype(jnp.int32)
            valid = ((tw >= 0.0) & (tw < 400.0) & (th >= 0.0)
                     & (th < 400.0) & (td >= 0.0) & (td < 12.0))
            pid = sp + g * 16 + iota
            tpl = tplb + jnp.where(pid >= thr, DHW, 0)
            flat = tpl + idd * HW + ih * W + iw
            mhigh = flat >= HALF
            off = flat + jnp.where(mhigh, PADW, 0)
            keep = valid & (mhigh == chigh)
            h = (off ^ lax.shift_right_logical(off, 13)) & (HTSZ - 1)
            got = plsc.load_gather(ht, [h], mask=keep)
            plsc.store_scatter(ht, [h], off, mask=keep)
            keep = keep & (got != off)
            plsc.store_compressed(sbuf.at[pl.ds(pos, 16)], off, mask=keep)
            pos = pos + jnp.sum(keep.astype(jnp.int32))
            return lax.cond(pos >= ROW, _flush, lambda a: a, (pos, row))

        def _grp2(g2, carry, _grp=_grp):
            return _grp(g2 * 2 + 1, _grp(g2 * 2, carry))

        with jax.named_scope("ph2_compute"):
            pos, row = lax.fori_loop(0, GROUPS // 2, _grp2, (pos, row))

    with jax.named_scope("ph2_tail"):
        for k in range(ROW // 16):
            sbuf[pl.ds(pos + k * 16, 16)] = trashv
        _, row = lax.cond(pos > 0,
                          lambda a: _flush(a),
                          lambda a: a, (pos, row))

    def _zdrain(k, c):
        pltpu.make_async_copy(zbuf, out.at[pl.ds(zbase + k * ZCH, ZCH)],
                              zsem).wait()
        return c

    with jax.named_scope("ph3_zdrain"):
        lax.fori_loop(0, ZN, _zdrain, 0)
    with jax.named_scope("ph4_barrier"):
        plsc.subcore_barrier()

    def _sfire(r, c):
        pltpu.async_copy(ones, out.at[ibuf.at[r]], ssem)
        return c

    with jax.named_scope("ph5_sfire"):
        lax.fori_loop(0, row, _sfire, 0)

    def _sdrain(r, c):
        pltpu.make_async_copy(ones, out.at[ibuf.at[0]], ssem).wait()
        return c

    with jax.named_scope("ph6_sdrain"):
        lax.fori_loop(0, row, _sdrain, 0)


@jax.jit
def _voxelize(xs, ys, zs):
    mesh = plsc.VectorSubcoreMesh(core_axis_name="c", subcore_axis_name="s")
    grid = pl.kernel(
        _body,
        out_type=jax.ShapeDtypeStruct((PGRID,), jnp.float32),
        mesh=mesh,
        compiler_params=pltpu.CompilerParams(needs_layout_passes=False),
        scratch_types=(
            [pltpu.VMEM((STAGE_PTS,), jnp.float32) for _ in range(6)]
            + [
                pltpu.VMEM((MAXROWS, ROW), jnp.int32),
                pltpu.VMEM((2 * ROW + 32,), jnp.int32),
                pltpu.VMEM((ZCH,), jnp.float32),
                pltpu.VMEM((ROW,), jnp.float32),
                pltpu.VMEM((HTSZ,), jnp.int32),
                pltpu.SemaphoreType.DMA,
                pltpu.SemaphoreType.DMA,
                pltpu.SemaphoreType.DMA,
            ]
        ),
    )(xs, ys, zs)
    lo = grid[:HALF]
    hi = grid[RSZ:RSZ + HALF]
    return jnp.maximum(
        jnp.concatenate([lo, hi]).reshape(T * D, H, W), 0.0)


def kernel(lidars):
    pts = lidars[0]
    xs = pts[:, :, 0].reshape(-1)
    ys = pts[:, :, 1].reshape(-1)
    zs = pts[:, :, 2].reshape(-1)
    return _voxelize(xs, ys, zs)


# speedup vs baseline: 3.3834x; 1.1513x over previous
import jax
import jax.numpy as jnp
from jax import lax
from jax.experimental import pallas as pl
from jax.experimental.pallas import tpu as pltpu
from jax.experimental.pallas import tpu_sc as plsc

W = 400
H = 400
D = 12
T = 5
HW = H * W
DHW = D * HW
NPTS = T * 100000
GRID = T * DHW
HALF = GRID // 2
PADW = 512
RSZ = HALF + PADW
PGRID = 2 * RSZ

NTILES = 16
TILE_PTS = 31256
STAGE_PTS = 2048
NSTAGES = 16
GROUPS = STAGE_PTS // 16
ROW = 128
MAXROWS = NSTAGES * STAGE_PTS // ROW + 2
ZCH = 18752
ZN = RSZ // NTILES // ZCH
HTSZ = 32768


def _body(xs_h, ys_h, zs_h, out, pbx0, pbx1, pby0, pby1, pbz0, pbz1,
          ibuf, sbuf, zbuf, ones, ht, zsem, psem, ssem):
    pbx = (pbx0, pbx1)
    pby = (pby0, pby1)
    pbz = (pbz0, pbz1)
    cid = lax.axis_index("c")
    wid = lax.axis_index("s")
    chigh = cid == 1
    iota = lax.iota(jnp.int32, 16)
    zvec = jnp.zeros((16,), jnp.float32)
    onev = jnp.full((16,), 1.0, jnp.float32)
    trashv = cid * RSZ + HALF + wid * 32 + iota

    def _zb(i, c):
        zbuf[pl.ds(i * 16, 16)] = zvec
        return c

    sent = jnp.full((16,), -1, jnp.int32)

    def _hti(i, c):
        for k in range(4):
            ht[pl.ds(i * 64 + k * 16, 16)] = sent
        return c

    with jax.named_scope("ph0_init"):
        lax.fori_loop(0, ZCH // 16, _zb, 0)

    zbase = cid * RSZ + wid * (RSZ // NTILES)

    def _zfire(k, c):
        pltpu.async_copy(zbuf, out.at[pl.ds(zbase + k * ZCH, ZCH)], zsem)
        return c

    with jax.named_scope("ph1_zfire"):
        lax.fori_loop(0, ZN, _zfire, 0)

    with jax.named_scope("ph1_init2"):
        for i in range(ROW // 16):
            ones[pl.ds(i * 16, 16)] = onev
        lax.fori_loop(0, HTSZ // 64, _hti, 0)

    base = wid * TILE_PTS

    def _sstart(s):
        return jnp.minimum(base + s * STAGE_PTS, NPTS - STAGE_PTS)

    def _pt_copies(s):
        sl = pl.ds(_sstart(s), STAGE_PTS)
        b = s % 2
        return (pltpu.make_async_copy(xs_h.at[sl], pbx[b], psem),
                pltpu.make_async_copy(ys_h.at[sl], pby[b], psem),
                pltpu.make_async_copy(zs_h.at[sl], pbz[b], psem))

    def _flush(args):
        pos, row = args
        for k in range(ROW // 16):
            ibuf[row, pl.ds(k * 16, 16)] = sbuf[pl.ds(k * 16, 16)]
        for k in range(4):
            sbuf[pl.ds(k * 16, 16)] = sbuf[pl.ds(ROW + k * 16, 16)]
        return pos - ROW, row + 1

    for cp in _pt_copies(0):
        cp.start()
    pos = jnp.int32(0)
    row = jnp.int32(0)
    for s in range(NSTAGES):
      with jax.named_scope("ph2_stage"):
        if s + 1 < NSTAGES:
            for cp in _pt_copies(s + 1):
                cp.start()
        with jax.named_scope("ph2_wait"):
            for cp in _pt_copies(s):
                cp.wait()
        bx, by, bz = pbx[s % 2], pby[s % 2], pbz[s % 2]
        sp = _sstart(s)
        t0 = ((sp >= 100000).astype(jnp.int32)
              + (sp >= 200000).astype(jnp.int32)
              + (sp >= 300000).astype(jnp.int32)
              + (sp >= 400000).astype(jnp.int32))
        tplb = t0 * DHW
        thr = t0 * 100000 + 100000

        def _grp(g, carry, s=s, bx=bx, by=by, bz=bz, sp=sp,
                 tplb=tplb, thr=thr):
            pos, row = carry
            o = pl.ds(g * 16, 16)
            x = bx[o]
            y = by[o]
            z = bz[o]
            tw = (x + 50.0) * 4.0
            th = (y + 50.0) * 4.0
            td = (z + 3.0) * 2.0
            iw = tw.astype(jnp.int32)
            ih = th.astype(jnp.int32)
            idd = td.astype(jnp.int32)
            valid = ((tw >= 0.0) & (tw < 400.0) & (th >= 0.0)
                     & (th < 400.0) & (td >= 0.0) & (td < 12.0))
            pid = sp + g * 16 + iota
            tpl = tplb + jnp.where(pid >= thr, DHW, 0)
            flat = tpl + idd * HW + ih * W + iw
            mhigh = flat >= HALF
            off = flat + jnp.where(mhigh, PADW, 0)
            keep = valid & (mhigh == chigh)
            h = (off ^ lax.shift_right_logical(off, 13)) & (HTSZ - 1)
            got = plsc.load_gather(ht, [h], mask=keep)
            plsc.store_scatter(ht, [h], off, mask=keep)
            keep = keep & (got != off)
            plsc.store_compressed(sbuf.at[pl.ds(pos, 16)], off, mask=keep)
            return pos + jnp.sum(keep.astype(jnp.int32)), row

        def _grp4(g4, carry, _grp=_grp):
            for k in range(4):
                carry = _grp(g4 * 4 + k, carry)
            return lax.cond(carry[0] >= ROW, _flush, lambda a: a, carry)

        with jax.named_scope("ph2_compute"):
            pos, row = lax.fori_loop(0, GROUPS // 4, _grp4, (pos, row))

    with jax.named_scope("ph2_tail"):
        for k in range(ROW // 16):
            sbuf[pl.ds(pos + k * 16, 16)] = trashv
        _, row = lax.cond(pos > 0,
                          lambda a: _flush(a),
                          lambda a: a, (pos, row))

    def _zdrain(k, c):
        pltpu.make_async_copy(zbuf, out.at[pl.ds(zbase + k * ZCH, ZCH)],
                              zsem).wait()
        return c

    with jax.named_scope("ph3_zdrain"):
        lax.fori_loop(0, ZN, _zdrain, 0)
    with jax.named_scope("ph4_barrier"):
        plsc.subcore_barrier()

    def _sfire(r, c):
        pltpu.async_copy(ones, out.at[ibuf.at[r]], ssem)
        return c

    with jax.named_scope("ph5_sfire"):
        lax.fori_loop(0, row, _sfire, 0)

    def _sdrain(r, c):
        pltpu.make_async_copy(ones, out.at[ibuf.at[0]], ssem).wait()
        return c

    with jax.named_scope("ph6_sdrain"):
        lax.fori_loop(0, row, _sdrain, 0)


@jax.jit
def _voxelize(xs, ys, zs):
    mesh = plsc.VectorSubcoreMesh(core_axis_name="c", subcore_axis_name="s")
    grid = pl.kernel(
        _body,
        out_type=jax.ShapeDtypeStruct((PGRID,), jnp.float32),
        mesh=mesh,
        compiler_params=pltpu.CompilerParams(needs_layout_passes=False),
        scratch_types=(
            [pltpu.VMEM((STAGE_PTS,), jnp.float32) for _ in range(6)]
            + [
                pltpu.VMEM((MAXROWS, ROW), jnp.int32),
                pltpu.VMEM((2 * ROW + 64,), jnp.int32),
                pltpu.VMEM((ZCH,), jnp.float32),
                pltpu.VMEM((ROW,), jnp.float32),
                pltpu.VMEM((HTSZ,), jnp.int32),
                pltpu.SemaphoreType.DMA,
                pltpu.SemaphoreType.DMA,
                pltpu.SemaphoreType.DMA,
            ]
        ),
    )(xs, ys, zs)
    lo = grid[:HALF]
    hi = grid[RSZ:RSZ + HALF]
    return jnp.maximum(
        jnp.concatenate([lo, hi]).reshape(T * D, H, W), 0.0)


def kernel(lidars):
    pts = lidars[0]
    xs = pts[:, :, 0].reshape(-1)
    ys = pts[:, :, 1].reshape(-1)
    zs = pts[:, :, 2].reshape(-1)
    return _voxelize(xs, ys, zs)
